# baseline (speedup 1.0000x reference)
import functools

import jax
import jax.numpy as jnp
from jax.experimental import pallas as pl
from jax.experimental.pallas import tpu as pltpu

_NUM_EXPERTS = 64
_LOSS_WEIGHT = 0.001
_Z_LOSS_WEIGHT = 0.0001


def _body(x_ref, idx_ref, out_ref, psum, csum, zsum, *, batch, top_k):
    i = pl.program_id(0)
    n = pl.num_programs(0)

    @pl.when(i == 0)
    def _init():
        psum[...] = jnp.zeros_like(psum)
        csum[...] = jnp.zeros_like(csum)
        zsum[...] = jnp.zeros_like(zsum)

    x = x_ref[...]
    m = jnp.max(x, axis=1, keepdims=True)
    e = jnp.exp(x - m)
    s = jnp.sum(e, axis=1, keepdims=True)
    lse = m + jnp.log(s)
    zsum[...] += jnp.sum(lse * lse)
    psum[...] += jnp.sum(e / s, axis=0, keepdims=True)

    idx = idx_ref[...]
    iota = jax.lax.broadcasted_iota(jnp.int32, (1, _NUM_EXPERTS), 1)
    oh = (idx[:, 0:1] == iota).astype(jnp.float32)
    for k in range(1, top_k):
        oh += (idx[:, k:k + 1] == iota).astype(jnp.float32)
    csum[...] += jnp.sum(oh, axis=0, keepdims=True)

    @pl.when(i == n - 1)
    def _fini():
        balance = (_NUM_EXPERTS * _LOSS_WEIGHT / (batch * batch * top_k)) * jnp.sum(
            psum[...] * csum[...])
        z = (_Z_LOSS_WEIGHT / batch) * jnp.sum(zsum[...])
        out_ref[...] = jnp.reshape(balance + z, (1, 1))


def kernel(router_logits, expert_indices):
    batch, experts = router_logits.shape
    top_k = expert_indices.shape[1]
    assert experts == _NUM_EXPERTS
    grid = 16
    rows = batch // grid
    out = pl.pallas_call(
        functools.partial(_body, batch=batch, top_k=top_k),
        grid=(grid,),
        in_specs=[
            pl.BlockSpec((rows, experts), lambda i: (i, 0)),
            pl.BlockSpec((rows, top_k), lambda i: (i, 0)),
        ],
        out_specs=pl.BlockSpec((1, 1), lambda i: (0, 0)),
        out_shape=jax.ShapeDtypeStruct((1, 1), jnp.float32),
        scratch_shapes=[
            pltpu.VMEM((1, _NUM_EXPERTS), jnp.float32),
            pltpu.VMEM((1, _NUM_EXPERTS), jnp.float32),
            pltpu.VMEM((1, 1), jnp.float32),
        ],
    )(router_logits, expert_indices.astype(jnp.int32))
    return out[0, 0]
